# baseline (speedup 1.0000x reference)
import functools

import jax
import jax.numpy as jnp
from jax import lax
from jax.experimental import pallas as pl
from jax.experimental.pallas import tpu as pltpu
from jax.experimental.pallas import tpu_sc as plsc

NC = 2
NS = 16
NW = NC * NS
LANES = 16
FPW = 4
NACT = 96 // FPW
CHUNK = 2048
IDXB = 128


def _body(idx_hbm, z_hbm, mean_hbm, logvar_hbm,
          v_hbm, mu_hbm, lv_hbm,
          idx_v, off_v, mu_g, lv_g, z_c, o_v, o_mu, o_lv,
          sem_m, sem_l):
    n_entries = mean_hbm.shape[0] // 96
    n_chunks = idx_hbm.shape[0] // CHUNK
    nper = FPW * CHUNK
    wid = lax.axis_index("s") * NC + lax.axis_index("c")

    @pl.when(wid < NACT)
    def _():
        f0 = wid * FPW
        for c in range(n_chunks):
            pltpu.sync_copy(idx_hbm.at[pl.ds(c * CHUNK, CHUNK)], idx_v)
            pltpu.sync_copy(z_hbm.at[c, pl.ds(f0, FPW)], z_c)

            def offb(j, carry):
                iv = idx_v[pl.ds(j * LANES, LANES)]
                for r in range(FPW):
                    off_v[pl.ds(r * CHUNK + j * LANES, LANES)] = (
                        iv + (f0 + r) * n_entries
                    )
                return carry

            lax.fori_loop(0, CHUNK // LANES, offb, 0)

            def fire(table, dst, sem):
                def go(k, carry):
                    sl = pl.ds(k * IDXB, IDXB)
                    pltpu.async_copy(table.at[off_v.at[sl]], dst.at[sl], sem)
                    return carry

                lax.fori_loop(0, nper // IDXB, go, 0)

            def drain(table, dst, sem):
                def go(k, carry):
                    sl0 = pl.ds(0, IDXB)
                    pltpu.make_async_copy(
                        table.at[off_v.at[sl0]], dst.at[sl0], sem).wait()
                    return carry

                lax.fori_loop(0, nper // IDXB, go, 0)

            fire(mean_hbm, mu_g, sem_m)
            fire(logvar_hbm, lv_g, sem_l)
            drain(mean_hbm, mu_g, sem_m)
            drain(logvar_hbm, lv_g, sem_l)

            def comp(j, carry):
                for r in range(FPW):
                    sl = pl.ds(r * CHUNK + j * LANES, LANES)
                    zsl = pl.ds(j * LANES, LANES)
                    mu16 = mu_g[sl]
                    lv16 = lv_g[sl]
                    z16 = z_c[r, zsl]
                    o_mu[r, zsl] = mu16
                    o_lv[r, zsl] = lv16
                    o_v[r, zsl] = mu16 + jnp.exp(lv16 * 0.5) * z16
                return carry

            lax.fori_loop(0, CHUNK // LANES, comp, 0)

            fsl = pl.ds(f0, FPW)
            pltpu.sync_copy(o_v, v_hbm.at[c, fsl])
            pltpu.sync_copy(o_mu, mu_hbm.at[c, fsl])
            pltpu.sync_copy(o_lv, lv_hbm.at[c, fsl])


@jax.jit
def _sc_embed(i1, z3, mean1, logvar1):
    B = i1.shape[0]
    n_chunks = B // CHUNK
    out = jax.ShapeDtypeStruct((n_chunks, 96, CHUNK), jnp.float32)
    run = functools.partial(
        pl.kernel,
        out_type=[out, out, out],
        mesh=plsc.VectorSubcoreMesh(core_axis_name="c", subcore_axis_name="s"),
        scratch_types=[
            pltpu.VMEM((CHUNK,), jnp.int32),
            pltpu.VMEM((FPW * CHUNK,), jnp.int32),
            pltpu.VMEM((FPW * CHUNK,), jnp.float32),
            pltpu.VMEM((FPW * CHUNK,), jnp.float32),
            pltpu.VMEM((FPW, CHUNK), jnp.float32),
            pltpu.VMEM((FPW, CHUNK), jnp.float32),
            pltpu.VMEM((FPW, CHUNK), jnp.float32),
            pltpu.VMEM((FPW, CHUNK), jnp.float32),
            pltpu.SemaphoreType.DMA,
            pltpu.SemaphoreType.DMA,
        ],
        compiler_params=pltpu.CompilerParams(use_tc_tiling_on_sc=False),
    )(_body)
    return run(i1, z3, mean1, logvar1)


def kernel(i, z, mean, logvar):
    B, W, L = z.shape
    D = W * L
    n = mean.shape[0]
    n_chunks = B // CHUNK
    m1 = mean.transpose(1, 2, 0).reshape(D * n)
    l1 = logvar.transpose(1, 2, 0).reshape(D * n)
    z3 = z.reshape(n_chunks, CHUNK, D).transpose(0, 2, 1)
    v3, mu3, lv3 = _sc_embed(i.astype(jnp.int32), z3, m1, l1)

    def back(t):
        return t.transpose(0, 2, 1).reshape(B, W, L)

    return (back(v3), back(mu3), back(lv3))


# speedup vs baseline: 4.1440x; 4.1440x over previous
import functools

import jax
import jax.numpy as jnp
from jax import lax
from jax.experimental import pallas as pl
from jax.experimental.pallas import tpu as pltpu
from jax.experimental.pallas import tpu_sc as plsc

NC = 2
NS = 16
NW = NC * NS
LANES = 16
CH = 128


def _body(idx_hbm, z_hbm, mean_hbm, logvar_hbm, v_hbm, mu_hbm, lv_hbm,
          idx_v, mu_v, lv_v, z_v, sem_mu, sem_lv, sem_z):
    D = mean_hbm.shape[1]
    n_chunks = idx_v.shape[0]
    wid = lax.axis_index("s") * NC + lax.axis_index("c")
    row0 = wid * n_chunks
    pltpu.sync_copy(idx_hbm.at[pl.ds(row0, n_chunks)], idx_v)
    for c in range(n_chunks):
        base = (row0 + c) * CH
        g_mu = pltpu.async_copy(mean_hbm.at[idx_v.at[c]], mu_v, sem_mu)
        g_lv = pltpu.async_copy(logvar_hbm.at[idx_v.at[c]], lv_v, sem_lv)
        g_z = pltpu.async_copy(z_hbm.at[pl.ds(base, CH)], z_v, sem_z)
        g_mu.wait()
        g_lv.wait()
        g_z.wait()
        o_mu = pltpu.async_copy(mu_v, mu_hbm.at[pl.ds(base, CH)], sem_mu)
        o_lv = pltpu.async_copy(lv_v, lv_hbm.at[pl.ds(base, CH)], sem_lv)

        def row_body(r, carry):
            for k in range(D // LANES):
                sl = pl.ds(k * LANES, LANES)
                z_v[r, sl] = mu_v[r, sl] + jnp.exp(lv_v[r, sl] * 0.5) * z_v[r, sl]
            return carry

        lax.fori_loop(0, CH, row_body, 0)
        o_mu.wait()
        o_lv.wait()
        pltpu.sync_copy(z_v, v_hbm.at[pl.ds(base, CH)])


@jax.jit
def _sc_embed(i2, z2, mean2, logvar2):
    B, D = z2.shape
    n_chunks = B // (NW * CH)
    run = functools.partial(
        pl.kernel,
        out_type=[jax.ShapeDtypeStruct((B, D), jnp.float32)] * 3,
        mesh=plsc.VectorSubcoreMesh(core_axis_name="c", subcore_axis_name="s"),
        scratch_types=[
            pltpu.VMEM((n_chunks, CH), jnp.int32),
            pltpu.VMEM((CH, D), jnp.float32),
            pltpu.VMEM((CH, D), jnp.float32),
            pltpu.VMEM((CH, D), jnp.float32),
            pltpu.SemaphoreType.DMA,
            pltpu.SemaphoreType.DMA,
            pltpu.SemaphoreType.DMA,
        ],
        compiler_params=pltpu.CompilerParams(use_tc_tiling_on_sc=False),
    )(_body)
    return run(i2, z2, mean2, logvar2)


def kernel(i, z, mean, logvar):
    B, W, L = z.shape
    N = mean.shape[0]
    D = W * L
    v2, mu2, lv2 = _sc_embed(
        i.astype(jnp.int32).reshape(B // CH, CH),
        z.reshape(B, D),
        mean.reshape(N, D),
        logvar.reshape(N, D),
    )
    shp = (B, W, L)
    return (v2.reshape(shp), mu2.reshape(shp), lv2.reshape(shp))
